# baseline (speedup 1.0000x reference)
import functools

import jax
import jax.numpy as jnp
from jax import lax
from jax.experimental import pallas as pl
from jax.experimental.pallas import tpu as pltpu
from jax.experimental.pallas import tpu_sc as plsc

N = 10000
E = 320000
D = 128
EPS = 1e-07

NP = 10240
EP = 327680
CW = 128
K0 = 112
K1 = 48
KMAX = 112
RPT = 640
RB = 1024


def _sc_segsum(r, src, dst):
    mesh = plsc.VectorSubcoreMesh(core_axis_name="c", subcore_axis_name="s")

    @functools.partial(
        pl.kernel,
        out_type=jax.ShapeDtypeStruct((2, NP, D), jnp.float32),
        mesh=mesh,
        scratch_types=[
            pltpu.VMEM((16, CW), jnp.int32),
            pltpu.VMEM((16, CW), jnp.int32),
            pltpu.VMEM((CW, D), jnp.float32),
            pltpu.VMEM((CW, D), jnp.float32),
            pltpu.VMEM_SHARED((NP, D), jnp.float32),
            pltpu.SemaphoreType.DMA,
            pltpu.SemaphoreType.DMA,
        ],
    )
    def k(r_hbm, src_hbm, dst_hbm, out_hbm, src_v, dst_v, rows_a, rows_b,
          aggr_sh, sem_a, sem_b):
        c = lax.axis_index("c")
        s = lax.axis_index("s")

        with jax.named_scope("zero_vmem"):
            zero16 = jnp.zeros((16,), jnp.float32)

            @pl.loop(0, CW)
            def _(i):
                @pl.loop(0, D, step=16)
                def _(j):
                    rows_a[i, pl.ds(j, 16)] = zero16

        with jax.named_scope("zero_spmem"):
            @pl.loop(0, RPT, step=CW)
            def _(k0):
                pltpu.sync_copy(rows_a, aggr_sh.at[pl.ds(s * RPT + k0, CW)])

        plsc.subcore_barrier()

        base = jnp.where(c == 0, s * K0, 16 * K0 + s * K1)

        def pipeline(K):
            @pl.loop(0, K, step=16)
            def _(b):
                pltpu.sync_copy(src_hbm.at[pl.ds(base + b, 16)], src_v)
                pltpu.sync_copy(dst_hbm.at[pl.ds(base + b, 16)], dst_v)
                pltpu.async_copy(r_hbm.at[src_v.at[0]], rows_a, sem_a)

                @pl.loop(0, 16, step=2)
                def _(j):
                    pltpu.async_copy(r_hbm.at[src_v.at[j + 1]], rows_b, sem_b)
                    pltpu.make_async_copy(r_hbm.at[src_v.at[j]], rows_a,
                                          sem_a).wait()
                    pltpu.sync_copy(rows_a, aggr_sh.at[dst_v.at[j]], add=True)

                    @pl.when(j + 2 < 16)
                    def _():
                        pltpu.async_copy(r_hbm.at[src_v.at[j + 2]], rows_a,
                                         sem_a)

                    pltpu.make_async_copy(r_hbm.at[src_v.at[j + 1]], rows_b,
                                          sem_b).wait()
                    pltpu.sync_copy(rows_b, aggr_sh.at[dst_v.at[j + 1]],
                                    add=True)

        with jax.named_scope("edge_loop"):
            @pl.when(c == 0)
            def _():
                pipeline(K0)

            @pl.when(c == 1)
            def _():
                pipeline(K1)

        with jax.named_scope("post_barrier"):
            plsc.subcore_barrier()

        with jax.named_scope("readout"):
            pltpu.sync_copy(aggr_sh.at[pl.ds(s * RPT, RPT)],
                            out_hbm.at[c, pl.ds(s * RPT, RPT)])

    return k(r, src, dst)


def _relu_eps_body(x_ref, r_ref):
    r_ref[...] = jnp.maximum(x_ref[...], 0.0) + EPS


def _relu_eps(xp):
    return pl.pallas_call(
        _relu_eps_body,
        grid=(NP // RB,),
        in_specs=[pl.BlockSpec((RB, D), lambda i: (i, 0))],
        out_specs=pl.BlockSpec((RB, D), lambda i: (i, 0)),
        out_shape=jax.ShapeDtypeStruct((NP, D), jnp.float32),
    )(xp)


def _update_body(agg_ref, h_ref, w_ref, b_ref, hn_ref, rn_ref):
    t = agg_ref[0] + agg_ref[1] + h_ref[...]
    hn = jnp.dot(t, w_ref[...], preferred_element_type=jnp.float32) + b_ref[...]
    hn_ref[...] = hn
    rn_ref[...] = jnp.maximum(hn, 0.0) + EPS


def _update(agg, h, W, b):
    return pl.pallas_call(
        _update_body,
        grid=(NP // RB,),
        in_specs=[
            pl.BlockSpec((2, RB, D), lambda i: (0, i, 0)),
            pl.BlockSpec((RB, D), lambda i: (i, 0)),
            pl.BlockSpec((D, D), lambda i: (0, 0)),
            pl.BlockSpec((1, D), lambda i: (0, 0)),
        ],
        out_specs=[
            pl.BlockSpec((RB, D), lambda i: (i, 0)),
            pl.BlockSpec((RB, D), lambda i: (i, 0)),
        ],
        out_shape=[
            jax.ShapeDtypeStruct((NP, D), jnp.float32),
            jax.ShapeDtypeStruct((NP, D), jnp.float32),
        ],
    )(agg, h, W, b)


def _final_body(agg_ref, h_ref, w_ref, b_ref, wo_ref, bo_ref, o_ref):
    t = agg_ref[0] + agg_ref[1] + h_ref[...]
    hn = jnp.dot(t, w_ref[...], preferred_element_type=jnp.float32) + b_ref[...]
    o_ref[...] = jnp.dot(hn, wo_ref[...],
                         preferred_element_type=jnp.float32) + bo_ref[...]


def _final(agg, h, W, b, Wo, bo):
    return pl.pallas_call(
        _final_body,
        grid=(NP // RB,),
        in_specs=[
            pl.BlockSpec((2, RB, D), lambda i: (0, i, 0)),
            pl.BlockSpec((RB, D), lambda i: (i, 0)),
            pl.BlockSpec((D, D), lambda i: (0, 0)),
            pl.BlockSpec((1, D), lambda i: (0, 0)),
            pl.BlockSpec((D, D), lambda i: (0, 0)),
            pl.BlockSpec((1, D), lambda i: (0, 0)),
        ],
        out_specs=pl.BlockSpec((RB, D), lambda i: (i, 0)),
        out_shape=jax.ShapeDtypeStruct((NP, D), jnp.float32),
    )(agg, h, W, b, Wo, bo)


def kernel(x, edge_index, W0, b0, W1, b1, W2, b2, W3, b3, Wo, bo):
    ei = edge_index.astype(jnp.int32)
    src = jnp.concatenate([ei[0], jnp.zeros((EP - E,), jnp.int32)])
    pad_dst = N + jnp.arange(EP - E, dtype=jnp.int32) % (NP - N)
    dst = jnp.concatenate([ei[1], pad_dst])
    src = src.reshape(EP // CW, CW)
    dst = dst.reshape(EP // CW, CW)

    xp = jnp.pad(x, ((0, NP - N), (0, 0)))

    h = xp
    r = _relu_eps(xp)
    for W, b in ((W0, b0), (W1, b1), (W2, b2)):
        agg = _sc_segsum(r, src, dst)
        h, r = _update(agg, h, W, b.reshape(1, D))
    agg = _sc_segsum(r, src, dst)
    out = _final(agg, h, W3, b3.reshape(1, D), Wo, bo.reshape(1, D))
    return out[:N]


# speedup vs baseline: 3.6610x; 3.6610x over previous
import functools

import jax
import jax.numpy as jnp
from jax import lax
from jax.experimental import pallas as pl
from jax.experimental.pallas import tpu as pltpu
from jax.experimental.pallas import tpu_sc as plsc

N = 10000
E = 320000
D = 128
EPS = 1e-07

NP = 10240
EP = 327680
CW = 128
K0 = 80
K1 = 80
RPT = 640
RB = 1024


def _sc_segsum(r, src, dst):
    mesh = plsc.VectorSubcoreMesh(core_axis_name="c", subcore_axis_name="s")

    @functools.partial(
        pl.kernel,
        out_type=jax.ShapeDtypeStruct((2, NP, D), jnp.float32),
        mesh=mesh,
        scratch_types=[
            pltpu.VMEM((16, CW), jnp.int32),
            pltpu.VMEM((16, CW), jnp.int32),
            pltpu.VMEM((CW, D), jnp.float32),
            pltpu.VMEM((CW, D), jnp.float32),
            pltpu.VMEM_SHARED((NP, D), jnp.float32),
            pltpu.SemaphoreType.DMA,
            pltpu.SemaphoreType.DMA,
        ],
    )
    def k(r_hbm, src_hbm, dst_hbm, out_hbm, src_v, dst_v, rows_a, rows_b,
          aggr_sh, sem_a, sem_b):
        c = lax.axis_index("c")
        s = lax.axis_index("s")

        with jax.named_scope("zero_vmem"):
            zero16 = jnp.zeros((16,), jnp.float32)

            @pl.loop(0, CW)
            def _(i):
                @pl.loop(0, D, step=16)
                def _(j):
                    rows_a[i, pl.ds(j, 16)] = zero16

        with jax.named_scope("zero_spmem"):
            @pl.loop(0, RPT, step=CW)
            def _(k0):
                pltpu.sync_copy(rows_a, aggr_sh.at[pl.ds(s * RPT + k0, CW)])

        plsc.subcore_barrier()

        base = jnp.where(c == 0, s * K0, 16 * K0 + s * K1)

        def pipeline(K):
            @pl.loop(0, K, step=16)
            def _(b):
                pltpu.sync_copy(src_hbm.at[pl.ds(base + b, 16)], src_v)
                pltpu.sync_copy(dst_hbm.at[pl.ds(base + b, 16)], dst_v)
                pltpu.async_copy(r_hbm.at[src_v.at[0]], rows_a, sem_a)

                @pl.loop(0, 16, step=2)
                def _(j):
                    pltpu.async_copy(r_hbm.at[src_v.at[j + 1]], rows_b, sem_b)
                    pltpu.make_async_copy(r_hbm.at[src_v.at[j]], rows_a,
                                          sem_a).wait()
                    pltpu.sync_copy(rows_a, aggr_sh.at[dst_v.at[j]], add=True)

                    @pl.when(j + 2 < 16)
                    def _():
                        pltpu.async_copy(r_hbm.at[src_v.at[j + 2]], rows_a,
                                         sem_a)

                    pltpu.make_async_copy(r_hbm.at[src_v.at[j + 1]], rows_b,
                                          sem_b).wait()
                    pltpu.sync_copy(rows_b, aggr_sh.at[dst_v.at[j + 1]],
                                    add=True)

        with jax.named_scope("edge_loop"):
            @pl.when(c == 0)
            def _():
                pipeline(K0)

            @pl.when(c == 1)
            def _():
                pipeline(K1)

        with jax.named_scope("post_barrier"):
            plsc.subcore_barrier()

        with jax.named_scope("readout"):
            pltpu.sync_copy(aggr_sh.at[pl.ds(s * RPT, RPT)],
                            out_hbm.at[c, pl.ds(s * RPT, RPT)])

    return k(r, src, dst)


def _relu_eps_body(x_ref, r_ref):
    r_ref[...] = jnp.maximum(x_ref[...], 0.0) + EPS


def _relu_eps(xp):
    return pl.pallas_call(
        _relu_eps_body,
        grid=(NP // RB,),
        in_specs=[pl.BlockSpec((RB, D), lambda i: (i, 0))],
        out_specs=pl.BlockSpec((RB, D), lambda i: (i, 0)),
        out_shape=jax.ShapeDtypeStruct((NP, D), jnp.float32),
    )(xp)


def _update_body(agg_ref, h_ref, w_ref, b_ref, hn_ref, rn_ref):
    t = agg_ref[0] + agg_ref[1] + h_ref[...]
    hn = jnp.dot(t, w_ref[...], preferred_element_type=jnp.float32) + b_ref[...]
    hn_ref[...] = hn
    rn_ref[...] = jnp.maximum(hn, 0.0) + EPS


def _update(agg, h, W, b):
    return pl.pallas_call(
        _update_body,
        grid=(NP // RB,),
        in_specs=[
            pl.BlockSpec((2, RB, D), lambda i: (0, i, 0)),
            pl.BlockSpec((RB, D), lambda i: (i, 0)),
            pl.BlockSpec((D, D), lambda i: (0, 0)),
            pl.BlockSpec((1, D), lambda i: (0, 0)),
        ],
        out_specs=[
            pl.BlockSpec((RB, D), lambda i: (i, 0)),
            pl.BlockSpec((RB, D), lambda i: (i, 0)),
        ],
        out_shape=[
            jax.ShapeDtypeStruct((NP, D), jnp.float32),
            jax.ShapeDtypeStruct((NP, D), jnp.float32),
        ],
    )(agg, h, W, b)


def _final_body(agg_ref, h_ref, w_ref, b_ref, wo_ref, bo_ref, o_ref):
    t = agg_ref[0] + agg_ref[1] + h_ref[...]
    hn = jnp.dot(t, w_ref[...], preferred_element_type=jnp.float32) + b_ref[...]
    o_ref[...] = jnp.dot(hn, wo_ref[...],
                         preferred_element_type=jnp.float32) + bo_ref[...]


def _final(agg, h, W, b, Wo, bo):
    return pl.pallas_call(
        _final_body,
        grid=(NP // RB,),
        in_specs=[
            pl.BlockSpec((2, RB, D), lambda i: (0, i, 0)),
            pl.BlockSpec((RB, D), lambda i: (i, 0)),
            pl.BlockSpec((D, D), lambda i: (0, 0)),
            pl.BlockSpec((1, D), lambda i: (0, 0)),
            pl.BlockSpec((D, D), lambda i: (0, 0)),
            pl.BlockSpec((1, D), lambda i: (0, 0)),
        ],
        out_specs=pl.BlockSpec((RB, D), lambda i: (i, 0)),
        out_shape=jax.ShapeDtypeStruct((NP, D), jnp.float32),
    )(agg, h, W, b, Wo, bo)


def kernel(x, edge_index, W0, b0, W1, b1, W2, b2, W3, b3, Wo, bo):
    ei = edge_index.astype(jnp.int32)
    pad_ar = jnp.arange(EP - E, dtype=jnp.int32)
    src = jnp.concatenate([ei[0], pad_ar * 997 % N])
    dst = jnp.concatenate([ei[1], N + pad_ar % (NP - N)])
    src = src.reshape(EP // CW, CW)
    dst = dst.reshape(EP // CW, CW)

    xp = jnp.pad(x, ((0, NP - N), (0, 0)))

    h = xp
    r = _relu_eps(xp)
    for W, b in ((W0, b0), (W1, b1), (W2, b2)):
        agg = _sc_segsum(r, src, dst)
        h, r = _update(agg, h, W, b.reshape(1, D))
    agg = _sc_segsum(r, src, dst)
    out = _final(agg, h, W3, b3.reshape(1, D), Wo, bo.reshape(1, D))
    return out[:N]


# speedup vs baseline: 3.7776x; 1.0318x over previous
import functools

import jax
import jax.numpy as jnp
from jax import lax
from jax.experimental import pallas as pl
from jax.experimental.pallas import tpu as pltpu
from jax.experimental.pallas import tpu_sc as plsc

N = 10000
E = 320000
D = 128
EPS = 1e-07

NP = 10240
CW = 128
NCH = E // CW
KW = 80
KLAST = NCH - 31 * KW
RPT = 640
RB = 1024


def _sc_segsum(r, edges):
    mesh = plsc.VectorSubcoreMesh(core_axis_name="c", subcore_axis_name="s")

    @functools.partial(
        pl.kernel,
        out_type=jax.ShapeDtypeStruct((2, NP, D), jnp.float32),
        mesh=mesh,
        scratch_types=[
            pltpu.VMEM((16, CW), jnp.int32),
            pltpu.VMEM((16, CW), jnp.int32),
            pltpu.VMEM((CW, D), jnp.float32),
            pltpu.VMEM((CW, D), jnp.float32),
            pltpu.VMEM_SHARED((NP, D), jnp.float32),
            pltpu.SemaphoreType.DMA,
            pltpu.SemaphoreType.DMA,
        ],
    )
    def k(r_hbm, e_hbm, out_hbm, src_v, dst_v, rows_a, rows_b,
          aggr_sh, sem_a, sem_b):
        c = lax.axis_index("c")
        s = lax.axis_index("s")

        with jax.named_scope("zero_vmem"):
            zero16 = jnp.zeros((16,), jnp.float32)

            @pl.loop(0, CW)
            def _(i):
                @pl.loop(0, D, step=16)
                def _(j):
                    rows_a[i, pl.ds(j, 16)] = zero16

        with jax.named_scope("zero_spmem"):
            @pl.loop(0, RPT, step=CW)
            def _(k0):
                pltpu.sync_copy(rows_a, aggr_sh.at[pl.ds(s * RPT + k0, CW)])

        plsc.subcore_barrier()

        w = c * 16 + s
        base = w * KW

        def batch(b0, B):
            pltpu.sync_copy(e_hbm.at[0, pl.ds(b0, B)], src_v.at[pl.ds(0, B)])
            pltpu.sync_copy(e_hbm.at[1, pl.ds(b0, B)], dst_v.at[pl.ds(0, B)])
            pltpu.async_copy(r_hbm.at[src_v.at[0]], rows_a, sem_a)

            @pl.loop(0, B, step=2)
            def _(j):
                pltpu.async_copy(r_hbm.at[src_v.at[j + 1]], rows_b, sem_b)
                pltpu.make_async_copy(r_hbm.at[src_v.at[j]], rows_a,
                                      sem_a).wait()
                pltpu.sync_copy(rows_a, aggr_sh.at[dst_v.at[j]], add=True)

                @pl.when(j + 2 < B)
                def _():
                    pltpu.async_copy(r_hbm.at[src_v.at[j + 2]], rows_a,
                                     sem_a)

                pltpu.make_async_copy(r_hbm.at[src_v.at[j + 1]], rows_b,
                                      sem_b).wait()
                pltpu.sync_copy(rows_b, aggr_sh.at[dst_v.at[j + 1]],
                                add=True)

        with jax.named_scope("edge_loop"):
            @pl.when(w < 31)
            def _():
                @pl.loop(0, KW, step=16)
                def _(b):
                    batch(base + b, 16)

            @pl.when(w == 31)
            def _():
                @pl.loop(0, KLAST - (KLAST % 16), step=16)
                def _(b):
                    batch(base + b, 16)
                if KLAST % 16:
                    batch(base + KLAST - (KLAST % 16), KLAST % 16)

        with jax.named_scope("post_barrier"):
            plsc.subcore_barrier()

        with jax.named_scope("readout"):
            pltpu.sync_copy(aggr_sh.at[pl.ds(s * RPT, RPT)],
                            out_hbm.at[c, pl.ds(s * RPT, RPT)])

    return k(r, edges)


def _relu_eps_body(x_ref, r_ref):
    r_ref[...] = jnp.maximum(x_ref[...], 0.0) + EPS


def _relu_eps(xp):
    return pl.pallas_call(
        _relu_eps_body,
        grid=(NP // RB,),
        in_specs=[pl.BlockSpec((RB, D), lambda i: (i, 0))],
        out_specs=pl.BlockSpec((RB, D), lambda i: (i, 0)),
        out_shape=jax.ShapeDtypeStruct((NP, D), jnp.float32),
    )(xp)


def _update_body(agg_ref, h_ref, w_ref, b_ref, hn_ref, rn_ref):
    t = agg_ref[0] + agg_ref[1] + h_ref[...]
    hn = jnp.dot(t, w_ref[...], preferred_element_type=jnp.float32) + b_ref[...]
    hn_ref[...] = hn
    rn_ref[...] = jnp.maximum(hn, 0.0) + EPS


def _update(agg, h, W, b):
    return pl.pallas_call(
        _update_body,
        grid=(NP // RB,),
        in_specs=[
            pl.BlockSpec((2, RB, D), lambda i: (0, i, 0)),
            pl.BlockSpec((RB, D), lambda i: (i, 0)),
            pl.BlockSpec((D, D), lambda i: (0, 0)),
            pl.BlockSpec((1, D), lambda i: (0, 0)),
        ],
        out_specs=[
            pl.BlockSpec((RB, D), lambda i: (i, 0)),
            pl.BlockSpec((RB, D), lambda i: (i, 0)),
        ],
        out_shape=[
            jax.ShapeDtypeStruct((NP, D), jnp.float32),
            jax.ShapeDtypeStruct((NP, D), jnp.float32),
        ],
    )(agg, h, W, b)


def _final_body(agg_ref, h_ref, w_ref, b_ref, wo_ref, bo_ref, o_ref):
    t = agg_ref[0] + agg_ref[1] + h_ref[...]
    hn = jnp.dot(t, w_ref[...], preferred_element_type=jnp.float32) + b_ref[...]
    o_ref[...] = jnp.dot(hn, wo_ref[...],
                         preferred_element_type=jnp.float32) + bo_ref[...]


def _final(agg, h, W, b, Wo, bo):
    return pl.pallas_call(
        _final_body,
        grid=(NP // RB,),
        in_specs=[
            pl.BlockSpec((2, RB, D), lambda i: (0, i, 0)),
            pl.BlockSpec((RB, D), lambda i: (i, 0)),
            pl.BlockSpec((D, D), lambda i: (0, 0)),
            pl.BlockSpec((1, D), lambda i: (0, 0)),
            pl.BlockSpec((D, D), lambda i: (0, 0)),
            pl.BlockSpec((1, D), lambda i: (0, 0)),
        ],
        out_specs=pl.BlockSpec((RB, D), lambda i: (i, 0)),
        out_shape=jax.ShapeDtypeStruct((N, D), jnp.float32),
    )(agg, h, W, b, Wo, bo)


def kernel(x, edge_index, W0, b0, W1, b1, W2, b2, W3, b3, Wo, bo):
    edges = edge_index.astype(jnp.int32).reshape(2, NCH, CW)

    xp = jnp.pad(x, ((0, NP - N), (0, 0)))

    h = xp
    r = _relu_eps(xp)
    for W, b in ((W0, b0), (W1, b1), (W2, b2)):
        agg = _sc_segsum(r, edges)
        h, r = _update(agg, h, W, b.reshape(1, D))
    agg = _sc_segsum(r, edges)
    return _final(agg, h, W3, b3.reshape(1, D), Wo, bo.reshape(1, D))


# speedup vs baseline: 3.8722x; 1.0251x over previous
import functools

import jax
import jax.numpy as jnp
from jax import lax
from jax.experimental import pallas as pl
from jax.experimental.pallas import tpu as pltpu
from jax.experimental.pallas import tpu_sc as plsc

N = 10000
E = 320000
D = 128
EPS = 1e-07

NP = 10240
CW = 128
NCH = E // CW
KW = 80
KLAST = NCH - 31 * KW
RPT = 640
RB = 2048
NB = (N + RB - 1) // RB


def _sc_segsum(r, edges):
    mesh = plsc.VectorSubcoreMesh(core_axis_name="c", subcore_axis_name="s")

    @functools.partial(
        pl.kernel,
        out_type=jax.ShapeDtypeStruct((2, NP, D), jnp.float32),
        mesh=mesh,
        scratch_types=[
            pltpu.VMEM((16, CW), jnp.int32),
            pltpu.VMEM((16, CW), jnp.int32),
            pltpu.VMEM((CW, D), jnp.float32),
            pltpu.VMEM((CW, D), jnp.float32),
            pltpu.VMEM_SHARED((NP, D), jnp.float32),
            pltpu.SemaphoreType.DMA,
            pltpu.SemaphoreType.DMA,
        ],
    )
    def k(r_hbm, e_hbm, out_hbm, src_v, dst_v, rows_a, rows_b,
          aggr_sh, sem_a, sem_b):
        c = lax.axis_index("c")
        s = lax.axis_index("s")

        with jax.named_scope("zero_vmem"):
            zero16 = jnp.zeros((16,), jnp.float32)

            @pl.loop(0, CW)
            def _(i):
                @pl.loop(0, D, step=16)
                def _(j):
                    rows_a[i, pl.ds(j, 16)] = zero16

        with jax.named_scope("zero_spmem"):
            @pl.loop(0, RPT, step=CW)
            def _(k0):
                pltpu.sync_copy(rows_a, aggr_sh.at[pl.ds(s * RPT + k0, CW)])

        plsc.subcore_barrier()

        w = c * 16 + s
        base = w * KW

        def batch(b0, B):
            pltpu.sync_copy(e_hbm.at[0, pl.ds(b0, B)], src_v.at[pl.ds(0, B)])
            pltpu.sync_copy(e_hbm.at[1, pl.ds(b0, B)], dst_v.at[pl.ds(0, B)])
            pltpu.async_copy(r_hbm.at[src_v.at[0]], rows_a, sem_a)

            @pl.loop(0, B, step=2)
            def _(j):
                pltpu.async_copy(r_hbm.at[src_v.at[j + 1]], rows_b, sem_b)
                pltpu.make_async_copy(r_hbm.at[src_v.at[j]], rows_a,
                                      sem_a).wait()
                pltpu.sync_copy(rows_a, aggr_sh.at[dst_v.at[j]], add=True)

                @pl.when(j + 2 < B)
                def _():
                    pltpu.async_copy(r_hbm.at[src_v.at[j + 2]], rows_a,
                                     sem_a)

                pltpu.make_async_copy(r_hbm.at[src_v.at[j + 1]], rows_b,
                                      sem_b).wait()
                pltpu.sync_copy(rows_b, aggr_sh.at[dst_v.at[j + 1]],
                                add=True)

        with jax.named_scope("edge_loop"):
            @pl.when(w < 31)
            def _():
                @pl.loop(0, KW, step=16)
                def _(b):
                    batch(base + b, 16)

            @pl.when(w == 31)
            def _():
                @pl.loop(0, KLAST - (KLAST % 16), step=16)
                def _(b):
                    batch(base + b, 16)
                if KLAST % 16:
                    batch(base + KLAST - (KLAST % 16), KLAST % 16)

        with jax.named_scope("post_barrier"):
            plsc.subcore_barrier()

        with jax.named_scope("readout"):
            pltpu.sync_copy(aggr_sh.at[pl.ds(s * RPT, RPT)],
                            out_hbm.at[c, pl.ds(s * RPT, RPT)])

    return k(r, edges)


def _relu_eps_body(x_ref, r_ref):
    r_ref[...] = jnp.maximum(x_ref[...], 0.0) + EPS


def _relu_eps(x):
    return pl.pallas_call(
        _relu_eps_body,
        grid=(NB,),
        in_specs=[pl.BlockSpec((RB, D), lambda i: (i, 0))],
        out_specs=pl.BlockSpec((RB, D), lambda i: (i, 0)),
        out_shape=jax.ShapeDtypeStruct((N, D), jnp.float32),
    )(x)


def _update_body(agg_ref, h_ref, w_ref, b_ref, hn_ref, rn_ref):
    t = agg_ref[0] + agg_ref[1] + h_ref[...]
    hn = jnp.dot(t, w_ref[...], preferred_element_type=jnp.float32) + b_ref[...]
    hn_ref[...] = hn
    rn_ref[...] = jnp.maximum(hn, 0.0) + EPS


def _update(agg, h, W, b):
    return pl.pallas_call(
        _update_body,
        grid=(NB,),
        in_specs=[
            pl.BlockSpec((2, RB, D), lambda i: (0, i, 0)),
            pl.BlockSpec((RB, D), lambda i: (i, 0)),
            pl.BlockSpec((D, D), lambda i: (0, 0)),
            pl.BlockSpec((1, D), lambda i: (0, 0)),
        ],
        out_specs=[
            pl.BlockSpec((RB, D), lambda i: (i, 0)),
            pl.BlockSpec((RB, D), lambda i: (i, 0)),
        ],
        out_shape=[
            jax.ShapeDtypeStruct((N, D), jnp.float32),
            jax.ShapeDtypeStruct((N, D), jnp.float32),
        ],
    )(agg, h, W, b)


def _final_body(agg_ref, h_ref, w_ref, b_ref, wo_ref, bo_ref, o_ref):
    t = agg_ref[0] + agg_ref[1] + h_ref[...]
    hn = jnp.dot(t, w_ref[...], preferred_element_type=jnp.float32) + b_ref[...]
    o_ref[...] = jnp.dot(hn, wo_ref[...],
                         preferred_element_type=jnp.float32) + bo_ref[...]


def _final(agg, h, W, b, Wo, bo):
    return pl.pallas_call(
        _final_body,
        grid=(NB,),
        in_specs=[
            pl.BlockSpec((2, RB, D), lambda i: (0, i, 0)),
            pl.BlockSpec((RB, D), lambda i: (i, 0)),
            pl.BlockSpec((D, D), lambda i: (0, 0)),
            pl.BlockSpec((1, D), lambda i: (0, 0)),
            pl.BlockSpec((D, D), lambda i: (0, 0)),
            pl.BlockSpec((1, D), lambda i: (0, 0)),
        ],
        out_specs=pl.BlockSpec((RB, D), lambda i: (i, 0)),
        out_shape=jax.ShapeDtypeStruct((N, D), jnp.float32),
    )(agg, h, W, b, Wo, bo)


def kernel(x, edge_index, W0, b0, W1, b1, W2, b2, W3, b3, Wo, bo):
    edges = edge_index.astype(jnp.int32).reshape(2, NCH, CW)

    h = x
    r = _relu_eps(x)
    for W, b in ((W0, b0), (W1, b1), (W2, b2)):
        agg = _sc_segsum(r, edges)
        h, r = _update(agg, h, W, b.reshape(1, D))
    agg = _sc_segsum(r, edges)
    return _final(agg, h, W3, b3.reshape(1, D), Wo, bo.reshape(1, D))


# speedup vs baseline: 4.1310x; 1.0668x over previous
import functools

import jax
import jax.numpy as jnp
from jax import lax
from jax.experimental import pallas as pl
from jax.experimental.pallas import tpu as pltpu
from jax.experimental.pallas import tpu_sc as plsc

N = 10000
E = 320000
D = 128
EPS = 1e-07

NP = 10240
CW = 128
NCH = E // CW
KW = 80
KLAST = NCH - 31 * KW
RPT = 640
RB = 2048
NB = (N + RB - 1) // RB


def _sc_segsum(r, edges):
    mesh = plsc.VectorSubcoreMesh(core_axis_name="c", subcore_axis_name="s")

    @functools.partial(
        pl.kernel,
        out_type=jax.ShapeDtypeStruct((2, NP, D), jnp.float32),
        mesh=mesh,
        scratch_types=[
            pltpu.VMEM((40, CW), jnp.int32),
            pltpu.VMEM((40, CW), jnp.int32),
            pltpu.VMEM((CW, D), jnp.float32),
            pltpu.VMEM((CW, D), jnp.float32),
            pltpu.VMEM_SHARED((NP, D), jnp.float32),
            pltpu.SemaphoreType.DMA,
            pltpu.SemaphoreType.DMA,
        ],
    )
    def k(r_hbm, e_hbm, out_hbm, src_v, dst_v, rows_a, rows_b,
          aggr_sh, sem_a, sem_b):
        c = lax.axis_index("c")
        s = lax.axis_index("s")

        with jax.named_scope("zero_vmem"):
            zero16 = jnp.zeros((16,), jnp.float32)

            @pl.loop(0, CW)
            def _(i):
                @pl.loop(0, D, step=16)
                def _(j):
                    rows_a[i, pl.ds(j, 16)] = zero16

        with jax.named_scope("zero_spmem"):
            @pl.loop(0, RPT, step=CW)
            def _(k0):
                pltpu.sync_copy(rows_a, aggr_sh.at[pl.ds(s * RPT + k0, CW)])

        plsc.subcore_barrier()

        w = c * 16 + s
        base = w * KW

        def batch(b0, B):
            pltpu.sync_copy(e_hbm.at[0, pl.ds(b0, B)], src_v.at[pl.ds(0, B)])
            pltpu.sync_copy(e_hbm.at[1, pl.ds(b0, B)], dst_v.at[pl.ds(0, B)])
            pltpu.async_copy(r_hbm.at[src_v.at[0]], rows_a, sem_a)

            @pl.loop(0, B, step=2)
            def _(j):
                pltpu.async_copy(r_hbm.at[src_v.at[j + 1]], rows_b, sem_b)
                pltpu.make_async_copy(r_hbm.at[src_v.at[j]], rows_a,
                                      sem_a).wait()
                pltpu.sync_copy(rows_a, aggr_sh.at[dst_v.at[j]], add=True)

                @pl.when(j + 2 < B)
                def _():
                    pltpu.async_copy(r_hbm.at[src_v.at[j + 2]], rows_a,
                                     sem_a)

                pltpu.make_async_copy(r_hbm.at[src_v.at[j + 1]], rows_b,
                                      sem_b).wait()
                pltpu.sync_copy(rows_b, aggr_sh.at[dst_v.at[j + 1]],
                                add=True)

        with jax.named_scope("edge_loop"):
            @pl.when(w < 31)
            def _():
                @pl.loop(0, KW, step=40)
                def _(b):
                    batch(base + b, 40)

            @pl.when(w == 31)
            def _():
                batch(base, 16)
                batch(base + 16, KLAST - 16)

        with jax.named_scope("post_barrier"):
            plsc.subcore_barrier()

        with jax.named_scope("readout"):
            pltpu.sync_copy(aggr_sh.at[pl.ds(s * RPT, RPT)],
                            out_hbm.at[c, pl.ds(s * RPT, RPT)])

    return k(r, edges)


def _relu_eps_body(x_ref, r_ref):
    r_ref[...] = jnp.maximum(x_ref[...], 0.0) + EPS


def _relu_eps(x):
    return pl.pallas_call(
        _relu_eps_body,
        grid=(NB,),
        in_specs=[pl.BlockSpec((RB, D), lambda i: (i, 0))],
        out_specs=pl.BlockSpec((RB, D), lambda i: (i, 0)),
        out_shape=jax.ShapeDtypeStruct((N, D), jnp.float32),
    )(x)


def _update_body(agg_ref, h_ref, w_ref, b_ref, hn_ref, rn_ref):
    t = agg_ref[0] + agg_ref[1] + h_ref[...]
    hn = jnp.dot(t, w_ref[...], preferred_element_type=jnp.float32) + b_ref[...]
    hn_ref[...] = hn
    rn_ref[...] = jnp.maximum(hn, 0.0) + EPS


def _update(agg, h, W, b):
    return pl.pallas_call(
        _update_body,
        grid=(NB,),
        in_specs=[
            pl.BlockSpec((2, RB, D), lambda i: (0, i, 0)),
            pl.BlockSpec((RB, D), lambda i: (i, 0)),
            pl.BlockSpec((D, D), lambda i: (0, 0)),
            pl.BlockSpec((1, D), lambda i: (0, 0)),
        ],
        out_specs=[
            pl.BlockSpec((RB, D), lambda i: (i, 0)),
            pl.BlockSpec((RB, D), lambda i: (i, 0)),
        ],
        out_shape=[
            jax.ShapeDtypeStruct((N, D), jnp.float32),
            jax.ShapeDtypeStruct((N, D), jnp.float32),
        ],
    )(agg, h, W, b)


def _final_body(agg_ref, h_ref, w_ref, b_ref, wo_ref, bo_ref, o_ref):
    t = agg_ref[0] + agg_ref[1] + h_ref[...]
    hn = jnp.dot(t, w_ref[...], preferred_element_type=jnp.float32) + b_ref[...]
    o_ref[...] = jnp.dot(hn, wo_ref[...],
                         preferred_element_type=jnp.float32) + bo_ref[...]


def _final(agg, h, W, b, Wo, bo):
    return pl.pallas_call(
        _final_body,
        grid=(NB,),
        in_specs=[
            pl.BlockSpec((2, RB, D), lambda i: (0, i, 0)),
            pl.BlockSpec((RB, D), lambda i: (i, 0)),
            pl.BlockSpec((D, D), lambda i: (0, 0)),
            pl.BlockSpec((1, D), lambda i: (0, 0)),
            pl.BlockSpec((D, D), lambda i: (0, 0)),
            pl.BlockSpec((1, D), lambda i: (0, 0)),
        ],
        out_specs=pl.BlockSpec((RB, D), lambda i: (i, 0)),
        out_shape=jax.ShapeDtypeStruct((N, D), jnp.float32),
    )(agg, h, W, b, Wo, bo)


def kernel(x, edge_index, W0, b0, W1, b1, W2, b2, W3, b3, Wo, bo):
    edges = edge_index.astype(jnp.int32).reshape(2, NCH, CW)

    h = x
    r = _relu_eps(x)
    for W, b in ((W0, b0), (W1, b1), (W2, b2)):
        agg = _sc_segsum(r, edges)
        h, r = _update(agg, h, W, b.reshape(1, D))
    agg = _sc_segsum(r, edges)
    return _final(agg, h, W3, b3.reshape(1, D), Wo, bo.reshape(1, D))
